# baseline (speedup 1.0000x reference)
import jax
import jax.numpy as jnp
from jax.experimental import pallas as pl

_BR = 8


def _masked_softmax_kernel(x_ref, m_ref, o_ref):
    x = x_ref[...]
    msk = m_ref[...]
    mx = jnp.max(x, axis=1, keepdims=True)
    e = jnp.exp(x - mx) * msk
    s = jnp.sum(e, axis=1, keepdims=True)
    o_ref[...] = e * (1.0 / s)


def kernel(input, mask):
    B, V = input.shape
    return pl.pallas_call(
        _masked_softmax_kernel,
        grid=(B // _BR,),
        in_specs=[
            pl.BlockSpec((_BR, V), lambda i: (i, 0)),
            pl.BlockSpec((_BR, V), lambda i: (i, 0)),
        ],
        out_specs=pl.BlockSpec((_BR, V), lambda i: (i, 0)),
        out_shape=jax.ShapeDtypeStruct((B, V), jnp.float32),
    )(input, mask)


# speedup vs baseline: 1.0288x; 1.0288x over previous
import jax
import jax.numpy as jnp
from jax.experimental import pallas as pl

_BR = 16


def _masked_softmax_kernel(x_ref, m_ref, o_ref):
    x = x_ref[...]
    msk = m_ref[...]
    mx = jnp.max(x, axis=1, keepdims=True)
    e = jnp.exp(x - mx) * msk
    s = jnp.sum(e, axis=1, keepdims=True)
    o_ref[...] = e * (1.0 / s)


def kernel(input, mask):
    B, V = input.shape
    return pl.pallas_call(
        _masked_softmax_kernel,
        grid=(B // _BR,),
        in_specs=[
            pl.BlockSpec((_BR, V), lambda i: (i, 0)),
            pl.BlockSpec((_BR, V), lambda i: (i, 0)),
        ],
        out_specs=pl.BlockSpec((_BR, V), lambda i: (i, 0)),
        out_shape=jax.ShapeDtypeStruct((B, V), jnp.float32),
    )(input, mask)
